# baseline (speedup 1.0000x reference)
import functools

import jax
import jax.numpy as jnp
from jax.experimental import pallas as pl
from jax.experimental.pallas import tpu as pltpu

_MAX_EXP = 15
_NVAL = 16
_NPOS = 16
_EMB = 32
_IN = _NPOS * _EMB
_HID = 256
_ACT = 4
_OH = _NPOS * _NVAL


def _body(bs, boards_ref, table_ref, w1_ref, b1_ref, w2_ref, b2_ref,
          w3_ref, b3_ref, out_ref, u_ref):
    @pl.when(pl.program_id(0) == 0)
    def _():
        t = table_ref[...]
        for p in range(_NPOS):
            u_ref[p * _NVAL:(p + 1) * _NVAL, :] = jnp.dot(
                t, w1_ref[p * _EMB:(p + 1) * _EMB, :],
                preferred_element_type=jnp.float32)

    enc = jnp.clip(boards_ref[...], 0, _MAX_EXP).astype(jnp.float32)

    rows = jax.lax.broadcasted_iota(jnp.int32, (_NPOS, _OH), 0)
    cols = jax.lax.broadcasted_iota(jnp.int32, (_NPOS, _OH), 1)
    rep = (cols // _NVAL == rows).astype(jnp.float32)
    encrep = jnp.dot(enc, rep, preferred_element_type=jnp.float32)

    vmod = (jax.lax.broadcasted_iota(jnp.int32, (bs, _OH), 1) % _NVAL
            ).astype(jnp.float32)
    oh = (encrep == vmod).astype(jnp.float32)

    h = jnp.dot(oh, u_ref[...], preferred_element_type=jnp.float32)
    h = jnp.maximum(h + b1_ref[...][None, :], 0.0)
    h = jnp.dot(h, w2_ref[...], preferred_element_type=jnp.float32)
    h = jnp.maximum(h + b2_ref[...][None, :], 0.0)
    out_ref[...] = (jnp.dot(h, w3_ref[...], preferred_element_type=jnp.float32)
                    + b3_ref[...][None, :])


@jax.jit
def kernel(boards, table, W1, b1, W2, b2, W3, b3):
    B = boards.shape[0]
    bs = 8192
    grid = B // bs

    out = pl.pallas_call(
        functools.partial(_body, bs),
        grid=(grid,),
        in_specs=[
            pl.BlockSpec((bs, _NPOS), lambda i: (i, 0)),
            pl.BlockSpec((_NVAL, _EMB), lambda i: (0, 0)),
            pl.BlockSpec((_IN, _HID), lambda i: (0, 0)),
            pl.BlockSpec((_HID,), lambda i: (0,)),
            pl.BlockSpec((_HID, _HID), lambda i: (0, 0)),
            pl.BlockSpec((_HID,), lambda i: (0,)),
            pl.BlockSpec((_HID, _ACT), lambda i: (0, 0)),
            pl.BlockSpec((_ACT,), lambda i: (0,)),
        ],
        out_specs=pl.BlockSpec((bs, _ACT), lambda i: (i, 0)),
        out_shape=jax.ShapeDtypeStruct((B, _ACT), jnp.float32),
        scratch_shapes=[pltpu.VMEM((_OH, _HID), jnp.float32)],
        compiler_params=pltpu.CompilerParams(
            dimension_semantics=("arbitrary",)),
    )(boards.astype(jnp.int32), table, W1, b1, W2, b2, W3, b3)
    return out


# speedup vs baseline: 1.0236x; 1.0236x over previous
import functools

import jax
import jax.numpy as jnp
from jax.experimental import pallas as pl
from jax.experimental.pallas import tpu as pltpu

_MAX_EXP = 15
_NVAL = 16
_NPOS = 16
_EMB = 32
_IN = _NPOS * _EMB
_HID = 256
_ACT = 4
_OH = _NPOS * _NVAL


def _body(bs, boards_ref, table_ref, w1_ref, b1_ref, w2_ref, b2_ref,
          w3_ref, b3_ref, out_ref, u_ref):
    @pl.when(pl.program_id(0) == 0)
    def _():
        t = table_ref[...]
        for p in range(_NPOS):
            u_ref[p * _NVAL:(p + 1) * _NVAL, :] = jnp.dot(
                t, w1_ref[p * _EMB:(p + 1) * _EMB, :],
                preferred_element_type=jnp.float32)

    enc = jnp.clip(boards_ref[...], 0, _MAX_EXP).astype(jnp.float32)

    rows = jax.lax.broadcasted_iota(jnp.int32, (_NPOS, _OH), 0)
    cols = jax.lax.broadcasted_iota(jnp.int32, (_NPOS, _OH), 1)
    rep = (cols // _NVAL == rows).astype(jnp.float32)
    encrep = jnp.dot(enc, rep, preferred_element_type=jnp.float32)

    vmod = (jax.lax.broadcasted_iota(jnp.int32, (bs, _OH), 1) % _NVAL
            ).astype(jnp.float32)
    oh = (encrep == vmod).astype(jnp.float32)

    h = jnp.dot(oh, u_ref[...], preferred_element_type=jnp.float32)
    h = jnp.maximum(h + b1_ref[...][None, :], 0.0)
    h = jnp.dot(h, w2_ref[...], preferred_element_type=jnp.float32)
    h = jnp.maximum(h + b2_ref[...][None, :], 0.0)
    out_ref[...] = (jnp.dot(h, w3_ref[...], preferred_element_type=jnp.float32)
                    + b3_ref[...][None, :])


@jax.jit
def kernel(boards, table, W1, b1, W2, b2, W3, b3):
    B = boards.shape[0]
    bs = 4096
    grid = B // bs

    out = pl.pallas_call(
        functools.partial(_body, bs),
        grid=(grid,),
        in_specs=[
            pl.BlockSpec((bs, _NPOS), lambda i: (i, 0)),
            pl.BlockSpec((_NVAL, _EMB), lambda i: (0, 0)),
            pl.BlockSpec((_IN, _HID), lambda i: (0, 0)),
            pl.BlockSpec((_HID,), lambda i: (0,)),
            pl.BlockSpec((_HID, _HID), lambda i: (0, 0)),
            pl.BlockSpec((_HID,), lambda i: (0,)),
            pl.BlockSpec((_HID, _ACT), lambda i: (0, 0)),
            pl.BlockSpec((_ACT,), lambda i: (0,)),
        ],
        out_specs=pl.BlockSpec((bs, _ACT), lambda i: (i, 0)),
        out_shape=jax.ShapeDtypeStruct((B, _ACT), jnp.float32),
        scratch_shapes=[pltpu.VMEM((_OH, _HID), jnp.float32)],
        compiler_params=pltpu.CompilerParams(
            dimension_semantics=("arbitrary",)),
    )(boards.astype(jnp.int32), table, W1, b1, W2, b2, W3, b3)
    return out


# speedup vs baseline: 1.3354x; 1.3046x over previous
import functools

import jax
import jax.numpy as jnp
from jax.experimental import layout as jlayout
from jax.experimental import pallas as pl
from jax.experimental.pallas import tpu as pltpu

_MAX_EXP = 15
_NVAL = 16
_NPOS = 16
_EMB = 32
_IN = _NPOS * _EMB
_HID = 256
_ACT = 4
_OH = _NPOS * _NVAL


def _body(bs, boards_ref, table_ref, w1_ref, b1_ref, w2_ref, b2_ref,
          w3_ref, b3_ref, out_ref, u_ref):
    @pl.when(pl.program_id(0) == 0)
    def _():
        t = table_ref[...]
        for p in range(_NPOS):
            u_ref[p * _NVAL:(p + 1) * _NVAL, :] = jnp.dot(
                t, w1_ref[p * _EMB:(p + 1) * _EMB, :],
                preferred_element_type=jnp.float32)

    enc_t = jnp.clip(boards_ref[...], 0, _MAX_EXP).astype(jnp.float32)
    enc = enc_t.T

    rows = jax.lax.broadcasted_iota(jnp.int32, (_NPOS, _OH), 0)
    cols = jax.lax.broadcasted_iota(jnp.int32, (_NPOS, _OH), 1)
    rep = (cols // _NVAL == rows).astype(jnp.float32)
    encrep = jnp.dot(enc, rep, preferred_element_type=jnp.float32)

    vmod = (jax.lax.broadcasted_iota(jnp.int32, (bs, _OH), 1) % _NVAL
            ).astype(jnp.float32)
    oh = (encrep == vmod).astype(jnp.float32)

    h = jnp.dot(oh, u_ref[...], preferred_element_type=jnp.float32)
    h = jnp.maximum(h + b1_ref[...][None, :], 0.0)
    h = jnp.dot(h, w2_ref[...], preferred_element_type=jnp.float32)
    h = jnp.maximum(h + b2_ref[...][None, :], 0.0)
    out_ref[...] = (jnp.dot(h, w3_ref[...], preferred_element_type=jnp.float32)
                    + b3_ref[...][None, :])


def _run(boards_t, table, W1, b1, W2, b2, W3, b3):
    B = boards_t.shape[1]
    bs = 4096
    grid = B // bs

    out = pl.pallas_call(
        functools.partial(_body, bs),
        grid=(grid,),
        in_specs=[
            pl.BlockSpec((_NPOS, bs), lambda i: (0, i)),
            pl.BlockSpec((_NVAL, _EMB), lambda i: (0, 0)),
            pl.BlockSpec((_IN, _HID), lambda i: (0, 0)),
            pl.BlockSpec((_HID,), lambda i: (0,)),
            pl.BlockSpec((_HID, _HID), lambda i: (0, 0)),
            pl.BlockSpec((_HID,), lambda i: (0,)),
            pl.BlockSpec((_HID, _ACT), lambda i: (0, 0)),
            pl.BlockSpec((_ACT,), lambda i: (0,)),
        ],
        out_specs=pl.BlockSpec((bs, _ACT), lambda i: (i, 0)),
        out_shape=jax.ShapeDtypeStruct((B, _ACT), jnp.float32),
        scratch_shapes=[pltpu.VMEM((_OH, _HID), jnp.float32)],
        compiler_params=pltpu.CompilerParams(
            dimension_semantics=("arbitrary",)),
    )(boards_t, table, W1, b1, W2, b2, W3, b3)
    return out


@functools.lru_cache(maxsize=None)
def _jitted(sharding):
    fmt = jlayout.Format(
        jlayout.Layout(major_to_minor=(0, 1), tiling=((8, 128),)), sharding)
    return jax.jit(_run, out_shardings=fmt)


def kernel(boards, table, W1, b1, W2, b2, W3, b3):
    bt = boards.astype(jnp.int32).T
    try:
        fn = _jitted(boards.sharding)
    except Exception:
        fn = jax.jit(_run)
    return fn(bt, table, W1, b1, W2, b2, W3, b3)


# speedup vs baseline: 2.2414x; 1.6784x over previous
import functools

import jax
import jax.numpy as jnp
from jax.experimental import pallas as pl
from jax.experimental.pallas import tpu as pltpu

_MAX_EXP = 15
_NVAL = 16
_NPOS = 16
_EMB = 32
_IN = _NPOS * _EMB
_HID = 256
_ACT = 4
_OH = _NPOS * _NVAL


def _body(bs, boards_ref, table_ref, w1_ref, b1_ref, w2_ref, b2_ref,
          w3_ref, b3_ref, out_ref, u_ref, u_t_ref, w2_t_ref, w3_t_ref,
          bc2_ref, bc3_ref):
    @pl.when(pl.program_id(0) == 0)
    def _():
        t = table_ref[...]
        b1r = b1_ref[...][None, :] * (1.0 / _NPOS)
        for p in range(_NPOS):
            u_ref[p * _NVAL:(p + 1) * _NVAL, :] = jnp.dot(
                t, w1_ref[p * _EMB:(p + 1) * _EMB, :],
                preferred_element_type=jnp.float32) + b1r
        u_t_ref[...] = u_ref[...].T
        w2_t_ref[...] = w2_ref[...].T
        w3_t_ref[...] = w3_ref[...].T
        bc2_ref[...] = b2_ref[...][None, :].T
        bc3_ref[...] = b3_ref[...][None, :].T

    enc_t = jnp.clip(boards_ref[...], 0, _MAX_EXP).astype(jnp.float32)

    rows_c = jax.lax.broadcasted_iota(jnp.int32, (_OH, _NPOS), 0) // _NVAL
    cols_p = jax.lax.broadcasted_iota(jnp.int32, (_OH, _NPOS), 1)
    rep_t = (rows_c == cols_p).astype(jnp.float32)
    encrep_t = jnp.dot(rep_t, enc_t, preferred_element_type=jnp.float32)

    vmod = (jax.lax.broadcasted_iota(jnp.int32, (_OH, bs), 0) % _NVAL
            ).astype(jnp.float32)
    oh_t = (encrep_t == vmod).astype(jnp.float32)

    h = jnp.maximum(
        jnp.dot(u_t_ref[...], oh_t, preferred_element_type=jnp.float32), 0.0)
    h = jnp.maximum(
        jnp.dot(w2_t_ref[...], h, preferred_element_type=jnp.float32)
        + bc2_ref[...], 0.0)
    out_ref[...] = (jnp.dot(w3_t_ref[...], h, preferred_element_type=jnp.float32)
                    + bc3_ref[...])


@jax.jit
def kernel(boards, table, W1, b1, W2, b2, W3, b3):
    B = boards.shape[0]
    bs = 4096
    grid = B // bs
    boards_t = boards.astype(jnp.int32).T

    out_t = pl.pallas_call(
        functools.partial(_body, bs),
        grid=(grid,),
        in_specs=[
            pl.BlockSpec((_NPOS, bs), lambda i: (0, i)),
            pl.BlockSpec((_NVAL, _EMB), lambda i: (0, 0)),
            pl.BlockSpec((_IN, _HID), lambda i: (0, 0)),
            pl.BlockSpec((_HID,), lambda i: (0,)),
            pl.BlockSpec((_HID, _HID), lambda i: (0, 0)),
            pl.BlockSpec((_HID,), lambda i: (0,)),
            pl.BlockSpec((_HID, _ACT), lambda i: (0, 0)),
            pl.BlockSpec((_ACT,), lambda i: (0,)),
        ],
        out_specs=pl.BlockSpec((_ACT, bs), lambda i: (0, i)),
        out_shape=jax.ShapeDtypeStruct((_ACT, B), jnp.float32),
        scratch_shapes=[
            pltpu.VMEM((_OH, _HID), jnp.float32),
            pltpu.VMEM((_HID, _OH), jnp.float32),
            pltpu.VMEM((_HID, _HID), jnp.float32),
            pltpu.VMEM((_ACT, _HID), jnp.float32),
            pltpu.VMEM((_HID, 1), jnp.float32),
            pltpu.VMEM((_ACT, 1), jnp.float32),
        ],
        compiler_params=pltpu.CompilerParams(
            dimension_semantics=("arbitrary",)),
    )(boards_t, table, W1, b1, W2, b2, W3, b3)
    return out_t.T


# speedup vs baseline: 2.5520x; 1.1386x over previous
import functools

import jax
import jax.numpy as jnp
from jax.experimental import pallas as pl
from jax.experimental.pallas import tpu as pltpu

_MAX_EXP = 15
_NVAL = 16
_NPOS = 16
_EMB = 32
_IN = _NPOS * _EMB
_HID = 256
_ACT = 4
_OH = _NPOS * _NVAL


def _body(bs, boards_ref, table_ref, w1_ref, b1_ref, w2_ref, b2_ref,
          w3_t_ref, b3_ref, out_ref, u_ref, u_t_ref, w2_t_ref,
          bc2_ref, bc3_ref):
    @pl.when(pl.program_id(0) == 0)
    def _():
        t = table_ref[...]
        b1r = b1_ref[...][None, :] * (1.0 / _NPOS)
        for p in range(_NPOS):
            u_ref[p * _NVAL:(p + 1) * _NVAL, :] = jnp.dot(
                t, w1_ref[p * _EMB:(p + 1) * _EMB, :],
                preferred_element_type=jnp.float32) + b1r
        u_t_ref[...] = u_ref[...].T
        w2_t_ref[...] = w2_ref[...].T
        bc2_ref[...] = b2_ref[...][None, :].T
        bc3_ref[...] = b3_ref[...][None, :].T

    enc_t = jnp.clip(boards_ref[...], 0, _MAX_EXP).astype(jnp.float32)

    rows_c = jax.lax.broadcasted_iota(jnp.int32, (_OH, _NPOS), 0) // _NVAL
    cols_p = jax.lax.broadcasted_iota(jnp.int32, (_OH, _NPOS), 1)
    rep_t = (rows_c == cols_p).astype(jnp.float32)
    encrep_t = jnp.dot(rep_t, enc_t, preferred_element_type=jnp.float32)

    vmod = (jax.lax.broadcasted_iota(jnp.int32, (_OH, bs), 0) % _NVAL
            ).astype(jnp.float32)
    oh_t = (encrep_t == vmod).astype(jnp.float32)

    h = jnp.maximum(
        jnp.dot(u_t_ref[...], oh_t, preferred_element_type=jnp.float32), 0.0)
    h = jnp.maximum(
        jnp.dot(w2_t_ref[...], h, preferred_element_type=jnp.float32)
        + bc2_ref[...], 0.0)
    out_ref[...] = (jnp.dot(w3_t_ref[...], h,
                            preferred_element_type=jnp.float32)
                    + bc3_ref[...])


@jax.jit
def kernel(boards, table, W1, b1, W2, b2, W3, b3):
    B = boards.shape[0]
    bs = 4096
    grid = B // bs
    boards_t = boards.astype(jnp.int32).T

    out_t = pl.pallas_call(
        functools.partial(_body, bs),
        grid=(grid,),
        in_specs=[
            pl.BlockSpec((_NPOS, bs), lambda i: (0, i)),
            pl.BlockSpec((_NVAL, _EMB), lambda i: (0, 0)),
            pl.BlockSpec((_IN, _HID), lambda i: (0, 0)),
            pl.BlockSpec((_HID,), lambda i: (0,)),
            pl.BlockSpec((_HID, _HID), lambda i: (0, 0)),
            pl.BlockSpec((_HID,), lambda i: (0,)),
            pl.BlockSpec((_ACT, _HID), lambda i: (0, 0)),
            pl.BlockSpec((_ACT,), lambda i: (0,)),
        ],
        out_specs=pl.BlockSpec((_ACT, bs), lambda i: (0, i)),
        out_shape=jax.ShapeDtypeStruct((_ACT, B), jnp.float32),
        scratch_shapes=[
            pltpu.VMEM((_OH, _HID), jnp.float32),
            pltpu.VMEM((_HID, _OH), jnp.float32),
            pltpu.VMEM((_HID, _HID), jnp.float32),
            pltpu.VMEM((_HID, 1), jnp.float32),
            pltpu.VMEM((_ACT, 1), jnp.float32),
        ],
        compiler_params=pltpu.CompilerParams(
            dimension_semantics=("arbitrary",)),
    )(boards_t, table, W1, b1, W2, b2, W3.T, b3)
    return out_t.T


# speedup vs baseline: 3.0229x; 1.1845x over previous
import functools

import jax
import jax.numpy as jnp
from jax.experimental import pallas as pl
from jax.experimental.pallas import tpu as pltpu

_MAX_EXP = 15
_NVAL = 16
_NPOS = 16
_EMB = 32
_IN = _NPOS * _EMB
_HID = 256
_ACT = 4
_OH = _NPOS * _NVAL


def _body(bs, boards_ref, table_ref, w1_ref, b1_ref, w2_ref, b2_ref,
          w3_t_ref, b3_ref, out_ref, u_ref, u_t_ref, w2_t_ref,
          bc2_ref, bc3_ref):
    @pl.when(pl.program_id(0) == 0)
    def _():
        t = table_ref[...]
        b1r = b1_ref[...][None, :] * (1.0 / _NPOS)
        for p in range(_NPOS):
            u_ref[p * _NVAL:(p + 1) * _NVAL, :] = jnp.dot(
                t, w1_ref[p * _EMB:(p + 1) * _EMB, :],
                preferred_element_type=jnp.float32) + b1r
        u_t_ref[...] = u_ref[...].T.astype(jnp.bfloat16)
        w2_t_ref[...] = w2_ref[...].T.astype(jnp.bfloat16)
        bc2_ref[...] = b2_ref[...][None, :].T
        bc3_ref[...] = b3_ref[...][None, :].T

    enc_t = jnp.clip(boards_ref[...], 0, _MAX_EXP)

    enc_rep = jnp.broadcast_to(enc_t[:, None, :],
                               (_NPOS, _NVAL, bs)).reshape(_OH, bs)
    vmod = jax.lax.broadcasted_iota(jnp.int32, (_OH, bs), 0) % _NVAL
    oh_t = (enc_rep == vmod).astype(jnp.bfloat16)

    h = jnp.maximum(
        jnp.dot(u_t_ref[...], oh_t, preferred_element_type=jnp.float32), 0.0)
    h = jnp.maximum(
        jnp.dot(w2_t_ref[...], h.astype(jnp.bfloat16),
                preferred_element_type=jnp.float32)
        + bc2_ref[...], 0.0)
    out_ref[...] = (jnp.dot(w3_t_ref[...].astype(jnp.bfloat16),
                            h.astype(jnp.bfloat16),
                            preferred_element_type=jnp.float32)
                    + bc3_ref[...])


@jax.jit
def kernel(boards, table, W1, b1, W2, b2, W3, b3):
    B = boards.shape[0]
    bs = 4096
    grid = B // bs
    boards_t = boards.astype(jnp.int32).T

    out_t = pl.pallas_call(
        functools.partial(_body, bs),
        grid=(grid,),
        in_specs=[
            pl.BlockSpec((_NPOS, bs), lambda i: (0, i)),
            pl.BlockSpec((_NVAL, _EMB), lambda i: (0, 0)),
            pl.BlockSpec((_IN, _HID), lambda i: (0, 0)),
            pl.BlockSpec((_HID,), lambda i: (0,)),
            pl.BlockSpec((_HID, _HID), lambda i: (0, 0)),
            pl.BlockSpec((_HID,), lambda i: (0,)),
            pl.BlockSpec((_ACT, _HID), lambda i: (0, 0)),
            pl.BlockSpec((_ACT,), lambda i: (0,)),
        ],
        out_specs=pl.BlockSpec((_ACT, bs), lambda i: (0, i)),
        out_shape=jax.ShapeDtypeStruct((_ACT, B), jnp.float32),
        scratch_shapes=[
            pltpu.VMEM((_OH, _HID), jnp.float32),
            pltpu.VMEM((_HID, _OH), jnp.bfloat16),
            pltpu.VMEM((_HID, _HID), jnp.bfloat16),
            pltpu.VMEM((_HID, 1), jnp.float32),
            pltpu.VMEM((_ACT, 1), jnp.float32),
        ],
        compiler_params=pltpu.CompilerParams(
            dimension_semantics=("arbitrary",)),
    )(boards_t, table, W1, b1, W2, b2, W3.T, b3)
    return out_t.T


# speedup vs baseline: 3.0332x; 1.0034x over previous
import functools

import jax
import jax.numpy as jnp
from jax.experimental import pallas as pl
from jax.experimental.pallas import tpu as pltpu

_MAX_EXP = 15
_NVAL = 16
_NPOS = 16
_EMB = 32
_IN = _NPOS * _EMB
_HID = 256
_ACT = 4
_OH = _NPOS * _NVAL


def _body(bs, boards_ref, table_ref, w1_ref, b1_ref, w2_ref, b2_ref,
          w3_t_ref, b3_ref, out_ref, u_ref, u_t_ref, w2_t_ref,
          bc2_ref, bc3_ref):
    @pl.when(pl.program_id(0) == 0)
    def _():
        t = table_ref[...]
        b1r = b1_ref[...][None, :] * (1.0 / _NPOS)
        for p in range(_NPOS):
            u_ref[p * _NVAL:(p + 1) * _NVAL, :] = jnp.dot(
                t, w1_ref[p * _EMB:(p + 1) * _EMB, :],
                preferred_element_type=jnp.float32) + b1r
        u_t_ref[...] = u_ref[...].T.astype(jnp.bfloat16)
        w2_t_ref[...] = w2_ref[...].T.astype(jnp.bfloat16)
        bc2_ref[...] = b2_ref[...][None, :].T
        bc3_ref[...] = b3_ref[...][None, :].T

    enc_t = jnp.clip(boards_ref[...], 0, _MAX_EXP)

    enc_rep = jnp.broadcast_to(enc_t[:, None, :],
                               (_NPOS, _NVAL, bs)).reshape(_OH, bs)
    vmod = jax.lax.broadcasted_iota(jnp.int32, (_OH, bs), 0) % _NVAL
    oh_t = (enc_rep == vmod).astype(jnp.bfloat16)

    h = jnp.dot(u_t_ref[...], oh_t, preferred_element_type=jnp.float32)
    h = jnp.maximum(h.astype(jnp.bfloat16), jnp.bfloat16(0.0))
    h = (jnp.dot(w2_t_ref[...], h, preferred_element_type=jnp.float32)
         + bc2_ref[...])
    h = jnp.maximum(h.astype(jnp.bfloat16), jnp.bfloat16(0.0))
    out_ref[...] = (jnp.dot(w3_t_ref[...].astype(jnp.bfloat16), h,
                            preferred_element_type=jnp.float32)
                    + bc3_ref[...])


@jax.jit
def kernel(boards, table, W1, b1, W2, b2, W3, b3):
    B = boards.shape[0]
    bs = 4096
    grid = B // bs
    boards_t = boards.astype(jnp.int32).T

    out_t = pl.pallas_call(
        functools.partial(_body, bs),
        grid=(grid,),
        in_specs=[
            pl.BlockSpec((_NPOS, bs), lambda i: (0, i)),
            pl.BlockSpec((_NVAL, _EMB), lambda i: (0, 0)),
            pl.BlockSpec((_IN, _HID), lambda i: (0, 0)),
            pl.BlockSpec((_HID,), lambda i: (0,)),
            pl.BlockSpec((_HID, _HID), lambda i: (0, 0)),
            pl.BlockSpec((_HID,), lambda i: (0,)),
            pl.BlockSpec((_ACT, _HID), lambda i: (0, 0)),
            pl.BlockSpec((_ACT,), lambda i: (0,)),
        ],
        out_specs=pl.BlockSpec((_ACT, bs), lambda i: (0, i)),
        out_shape=jax.ShapeDtypeStruct((_ACT, B), jnp.float32),
        scratch_shapes=[
            pltpu.VMEM((_OH, _HID), jnp.float32),
            pltpu.VMEM((_HID, _OH), jnp.bfloat16),
            pltpu.VMEM((_HID, _HID), jnp.bfloat16),
            pltpu.VMEM((_HID, 1), jnp.float32),
            pltpu.VMEM((_ACT, 1), jnp.float32),
        ],
        compiler_params=pltpu.CompilerParams(
            dimension_semantics=("arbitrary",)),
    )(boards_t, table, W1, b1, W2, b2, W3.T, b3)
    return out_t.T


# speedup vs baseline: 3.1360x; 1.0339x over previous
import functools

import jax
import jax.numpy as jnp
from jax.experimental import pallas as pl
from jax.experimental.pallas import tpu as pltpu

_MAX_EXP = 15
_NVAL = 16
_NPOS = 16
_EMB = 32
_IN = _NPOS * _EMB
_HID = 256
_ACT = 4
_OH = _NPOS * _NVAL


def _body(bs, boards_ref, table_ref, w1_ref, b1_ref, w2_ref, b2_ref,
          w3_t_ref, b3_ref, out_ref, u_ref, u_t_ref, w2_t_ref,
          bc2_ref, bc3_ref):
    @pl.when(pl.program_id(0) == 0)
    def _():
        t = table_ref[...]
        b1r = b1_ref[...][None, :] * (1.0 / _NPOS)
        for p in range(_NPOS):
            u_ref[p * _NVAL:(p + 1) * _NVAL, :] = jnp.dot(
                t, w1_ref[p * _EMB:(p + 1) * _EMB, :],
                preferred_element_type=jnp.float32) + b1r
        u_t_ref[...] = u_ref[...].T.astype(jnp.bfloat16)
        w2_t_ref[...] = w2_ref[...].T.astype(jnp.bfloat16)
        bc2_ref[...] = b2_ref[...][None, :].T
        bc3_ref[...] = b3_ref[...][None, :].T

    enc_t = jnp.clip(boards_ref[...], 0, _MAX_EXP)

    enc_rep = jnp.broadcast_to(enc_t[:, None, :],
                               (_NPOS, _NVAL, bs)).reshape(_OH, bs)
    vmod = jax.lax.broadcasted_iota(jnp.int32, (_OH, bs), 0) % _NVAL
    oh_t = (enc_rep == vmod).astype(jnp.bfloat16)

    h = jnp.dot(u_t_ref[...], oh_t, preferred_element_type=jnp.float32)
    h = jnp.maximum(h.astype(jnp.bfloat16), jnp.bfloat16(0.0))
    h = (jnp.dot(w2_t_ref[...], h, preferred_element_type=jnp.float32)
         + bc2_ref[...])
    h = jnp.maximum(h.astype(jnp.bfloat16), jnp.bfloat16(0.0))
    out_ref[...] = (jnp.dot(w3_t_ref[...].astype(jnp.bfloat16), h,
                            preferred_element_type=jnp.float32)
                    + bc3_ref[...])


@jax.jit
def kernel(boards, table, W1, b1, W2, b2, W3, b3):
    B = boards.shape[0]
    bs = 8192
    grid = B // bs
    boards_t = boards.astype(jnp.int32).T

    out_t = pl.pallas_call(
        functools.partial(_body, bs),
        grid=(grid,),
        in_specs=[
            pl.BlockSpec((_NPOS, bs), lambda i: (0, i)),
            pl.BlockSpec((_NVAL, _EMB), lambda i: (0, 0)),
            pl.BlockSpec((_IN, _HID), lambda i: (0, 0)),
            pl.BlockSpec((_HID,), lambda i: (0,)),
            pl.BlockSpec((_HID, _HID), lambda i: (0, 0)),
            pl.BlockSpec((_HID,), lambda i: (0,)),
            pl.BlockSpec((_ACT, _HID), lambda i: (0, 0)),
            pl.BlockSpec((_ACT,), lambda i: (0,)),
        ],
        out_specs=pl.BlockSpec((_ACT, bs), lambda i: (0, i)),
        out_shape=jax.ShapeDtypeStruct((_ACT, B), jnp.float32),
        scratch_shapes=[
            pltpu.VMEM((_OH, _HID), jnp.float32),
            pltpu.VMEM((_HID, _OH), jnp.bfloat16),
            pltpu.VMEM((_HID, _HID), jnp.bfloat16),
            pltpu.VMEM((_HID, 1), jnp.float32),
            pltpu.VMEM((_ACT, 1), jnp.float32),
        ],
        compiler_params=pltpu.CompilerParams(
            dimension_semantics=("arbitrary",)),
    )(boards_t, table, W1, b1, W2, b2, W3.T, b3)
    return out_t.T


# speedup vs baseline: 3.2481x; 1.0357x over previous
import functools

import jax
import jax.numpy as jnp
from jax.experimental import pallas as pl
from jax.experimental.pallas import tpu as pltpu

_MAX_EXP = 15
_NVAL = 16
_NPOS = 16
_EMB = 32
_IN = _NPOS * _EMB
_HID = 256
_ACT = 4
_OH = _NPOS * _NVAL


def _body(bs, boards_ref, table_ref, w1_ref, b1_ref, w2_ref, b2_ref,
          w3_t_ref, b3_ref, out_ref, u_ref, u_t_ref, w2_t_ref,
          bc2_ref, bc3_ref):
    @pl.when(pl.program_id(0) == 0)
    def _():
        t = table_ref[...]
        b1r = b1_ref[...][None, :] * (1.0 / _NPOS)
        for p in range(_NPOS):
            u_ref[p * _NVAL:(p + 1) * _NVAL, :] = jnp.dot(
                t, w1_ref[p * _EMB:(p + 1) * _EMB, :],
                preferred_element_type=jnp.float32) + b1r
        u_t_ref[...] = u_ref[...].T.astype(jnp.bfloat16)
        w2_t_ref[...] = w2_ref[...].T.astype(jnp.bfloat16)
        bc2_ref[...] = b2_ref[...][None, :].T
        bc3_ref[...] = b3_ref[...][None, :].T

    enc_t = jnp.clip(boards_ref[...], 0, _MAX_EXP)

    enc_rep = jnp.broadcast_to(enc_t[:, None, :],
                               (_NPOS, _NVAL, bs)).reshape(_OH, bs)
    vmod = jax.lax.broadcasted_iota(jnp.int32, (_OH, bs), 0) % _NVAL
    oh_t = (enc_rep == vmod).astype(jnp.bfloat16)

    h = jnp.dot(u_t_ref[...], oh_t, preferred_element_type=jnp.float32)
    h = jnp.maximum(h.astype(jnp.bfloat16), jnp.bfloat16(0.0))
    h = (jnp.dot(w2_t_ref[...], h, preferred_element_type=jnp.float32)
         + bc2_ref[...])
    h = jnp.maximum(h.astype(jnp.bfloat16), jnp.bfloat16(0.0))
    out_ref[...] = (jnp.dot(w3_t_ref[...].astype(jnp.bfloat16), h,
                            preferred_element_type=jnp.float32)
                    + bc3_ref[...])


@jax.jit
def kernel(boards, table, W1, b1, W2, b2, W3, b3):
    B = boards.shape[0]
    bs = 16384
    grid = B // bs
    boards_t = boards.astype(jnp.int32).T

    out_t = pl.pallas_call(
        functools.partial(_body, bs),
        grid=(grid,),
        in_specs=[
            pl.BlockSpec((_NPOS, bs), lambda i: (0, i)),
            pl.BlockSpec((_NVAL, _EMB), lambda i: (0, 0)),
            pl.BlockSpec((_IN, _HID), lambda i: (0, 0)),
            pl.BlockSpec((_HID,), lambda i: (0,)),
            pl.BlockSpec((_HID, _HID), lambda i: (0, 0)),
            pl.BlockSpec((_HID,), lambda i: (0,)),
            pl.BlockSpec((_ACT, _HID), lambda i: (0, 0)),
            pl.BlockSpec((_ACT,), lambda i: (0,)),
        ],
        out_specs=pl.BlockSpec((_ACT, bs), lambda i: (0, i)),
        out_shape=jax.ShapeDtypeStruct((_ACT, B), jnp.float32),
        scratch_shapes=[
            pltpu.VMEM((_OH, _HID), jnp.float32),
            pltpu.VMEM((_HID, _OH), jnp.bfloat16),
            pltpu.VMEM((_HID, _HID), jnp.bfloat16),
            pltpu.VMEM((_HID, 1), jnp.float32),
            pltpu.VMEM((_ACT, 1), jnp.float32),
        ],
        compiler_params=pltpu.CompilerParams(
            dimension_semantics=("arbitrary",)),
    )(boards_t, table, W1, b1, W2, b2, W3.T, b3)
    return out_t.T
